# baseline (speedup 1.0000x reference)
import jax
import jax.numpy as jnp
from jax import lax
from jax.experimental import pallas as pl
from jax.experimental.pallas import tpu as pltpu
from jax.experimental.pallas import tpu_sc as plsc

BATCH = 16384
DIMS = 64
LANES = 16
TBL_ROWS = 1000000
FROWS = DIMS * TBL_ROWS // 8

_info = plsc.get_sparse_core_info()
NC = _info.num_cores
NS = _info.num_subcores
NW = NC * NS

B_PER_W = BATCH // NW
CHUNK = 32
N_CHUNKS = B_PER_W // CHUNK
ROWS_PER_B = 2 * DIMS


def _gmf_body(u_hbm, m_hbm, tabt_hbm, out_hbm,
              idx_u, idx_m, q_u, q_m, pos_u, pos_m, offs, vals, outb, sem):
    wid = lax.axis_index("s") * NC + lax.axis_index("c")
    base = pl.multiple_of(wid * B_PER_W, B_PER_W)

    tab8 = tabt_hbm

    pltpu.sync_copy(u_hbm.at[pl.ds(base, B_PER_W)], idx_u)
    pltpu.sync_copy(m_hbm.at[pl.ds(base, B_PER_W)], idx_m)

    def derive(c, _):
        sl = pl.ds(c * LANES, LANES)
        for ref, qref, pref in ((idx_u, q_u, pos_u), (idx_m, q_m, pos_m)):
            v = ref[sl]
            qref[sl] = lax.shift_right_logical(v, 3)
            pref[sl] = jnp.bitwise_and(v, 7)
        return 0

    lax.fori_loop(0, B_PER_W // LANES, derive, 0, unroll=4)

    iota = lax.iota(jnp.int32, LANES)
    cj = [(iota + g * LANES) * (TBL_ROWS // 8) for g in range(DIMS // LANES)]

    for k in range(N_CHUNKS):
        def build(bl, _):
            bvec = jnp.full((LANES,), k * CHUNK + bl, jnp.int32)
            s_u = plsc.load_gather(q_u, [bvec])
            s_m = plsc.load_gather(q_m, [bvec])
            for g in range(DIMS // LANES):
                offs[pl.ds(bl * ROWS_PER_B + g * LANES, LANES)] = s_u + cj[g]
                offs[pl.ds(bl * ROWS_PER_B + DIMS + g * LANES, LANES)] = \
                    s_m + cj[g]
            return 0

        lax.fori_loop(0, CHUNK, build, 0)

        pltpu.async_copy(tab8.at[offs], vals, sem).wait()

        def mul_vrow(v, _):
            for p in range(2):
                b = 2 * v + p
                gb = jnp.full((LANES,), k * CHUNK + b, jnp.int32)
                pu = plsc.load_gather(pos_u, [gb])
                pm = plsc.load_gather(pos_m, [gb])
                rbase = b * ROWS_PER_B
                for g in range(DIMS // LANES):
                    ru = jnp.full((LANES,), rbase + g * LANES, jnp.int32) + iota
                    a = plsc.load_gather(vals, [ru, pu])
                    c = plsc.load_gather(vals, [ru + DIMS, pm])
                    outb[v, pl.ds(p * DIMS + g * LANES, LANES)] = a * c
            return 0

        lax.fori_loop(0, CHUNK // 2, mul_vrow, 0)

        pltpu.sync_copy(
            outb,
            out_hbm.at[pl.ds(pl.multiple_of((base + k * CHUNK) // 2, CHUNK // 2),
                             CHUNK // 2)])


@jax.jit
def _gmf(u, m, user_table):
    tabt = jnp.transpose(user_table).reshape(FROWS, 8)
    kfn = pl.kernel(
        _gmf_body,
        out_type=jax.ShapeDtypeStruct((BATCH // 2, 2 * DIMS), jnp.float32),
        mesh=plsc.VectorSubcoreMesh(core_axis_name="c", subcore_axis_name="s"),
        compiler_params=pltpu.CompilerParams(
            use_tc_tiling_on_sc=False,
            needs_layout_passes=False,
        ),
        scratch_types=[
            pltpu.VMEM((B_PER_W,), jnp.int32),
            pltpu.VMEM((B_PER_W,), jnp.int32),
            pltpu.VMEM((B_PER_W,), jnp.int32),
            pltpu.VMEM((B_PER_W,), jnp.int32),
            pltpu.VMEM((B_PER_W,), jnp.int32),
            pltpu.VMEM((B_PER_W,), jnp.int32),
            pltpu.VMEM((CHUNK * ROWS_PER_B,), jnp.int32),
            pltpu.VMEM((CHUNK * ROWS_PER_B, 8), jnp.float32),
            pltpu.VMEM((CHUNK // 2, 2 * DIMS), jnp.float32),
            pltpu.SemaphoreType.DMA,
        ],
    )
    packed = kfn(u, m, tabt)
    return packed.reshape(BATCH, DIMS)


def kernel(u, m, user_table, movie_table):
    return _gmf(u, m, user_table)


# speedup vs baseline: 11.6753x; 11.6753x over previous
import jax
import jax.numpy as jnp
from jax import lax
from jax.experimental import pallas as pl
from jax.experimental.pallas import tpu as pltpu
from jax.experimental.pallas import tpu_sc as plsc

BATCH = 16384
DIMS = 64
LANES = 16
VROW = 2 * DIMS

_info = plsc.get_sparse_core_info()
NC = _info.num_cores
NS = _info.num_subcores
NW = NC * NS

B_PER_W = BATCH // NW
CHUNK = 32
N_CHUNKS = B_PER_W // CHUNK


def _gmf_body(u_hbm, m_hbm, table_hbm, out_hbm,
              idx_u, idx_m, blk_u, blk_m, outb, sem_u, sem_m):
    wid = lax.axis_index("s") * NC + lax.axis_index("c")
    base = pl.multiple_of(wid * B_PER_W, B_PER_W)

    pltpu.sync_copy(u_hbm.at[pl.ds(base, B_PER_W)], idx_u)
    pltpu.sync_copy(m_hbm.at[pl.ds(base, B_PER_W)], idx_m)

    iota = lax.iota(jnp.int32, LANES)

    def chunk_body(k, _):
        def issue(c16, _):
            vu = idx_u[pl.ds(k * CHUNK + c16 * LANES, LANES)]
            vm = idx_m[pl.ds(k * CHUNK + c16 * LANES, LANES)]
            for l in range(LANES):
                c = c16 * LANES + l
                ru = pl.multiple_of(
                    lax.shift_left(lax.shift_right_logical(vu[l], 3), 3), 8)
                rm = pl.multiple_of(
                    lax.shift_left(lax.shift_right_logical(vm[l], 3), 3), 8)
                pltpu.async_copy(
                    table_hbm.at[pl.ds(ru, 8), :], blk_u.at[c], sem_u)
                pltpu.async_copy(
                    table_hbm.at[pl.ds(rm, 8), :], blk_m.at[c], sem_m)
            return 0

        lax.fori_loop(0, CHUNK // LANES, issue, 0)

        def drain(c, _):
            pltpu.make_async_copy(
                table_hbm.at[pl.ds(0, 8), :], blk_u.at[c], sem_u).wait()
            pltpu.make_async_copy(
                table_hbm.at[pl.ds(0, 8), :], blk_m.at[c], sem_m).wait()
            return 0

        lax.fori_loop(0, CHUNK, drain, 0)

        def mul_vrow(v, _):
            for p in range(2):
                b = 2 * v + p
                gb = jnp.full((LANES,), k * CHUNK + b, jnp.int32)
                su = jnp.bitwise_and(plsc.load_gather(idx_u, [gb]), 7)
                sm = jnp.bitwise_and(plsc.load_gather(idx_m, [gb]), 7)
                bvec = jnp.full((LANES,), b, jnp.int32)
                for g in range(DIMS // LANES):
                    cols = iota + (g * LANES)
                    a = plsc.load_gather(blk_u, [bvec, su, cols])
                    c2 = plsc.load_gather(blk_m, [bvec, sm, cols])
                    outb[v, pl.ds(p * DIMS + g * LANES, LANES)] = a * c2
            return 0

        lax.fori_loop(0, CHUNK // 2, mul_vrow, 0)

        pltpu.sync_copy(
            outb,
            out_hbm.at[pl.ds(pl.multiple_of((base + k * CHUNK) // 2, CHUNK // 2),
                             CHUNK // 2)])
        return 0

    lax.fori_loop(0, N_CHUNKS, chunk_body, 0)


@jax.jit
def _gmf(u, m, user_table):
    kfn = pl.kernel(
        _gmf_body,
        out_type=jax.ShapeDtypeStruct((BATCH // 2, VROW), jnp.float32),
        mesh=plsc.VectorSubcoreMesh(core_axis_name="c", subcore_axis_name="s"),
        compiler_params=pltpu.CompilerParams(
            use_tc_tiling_on_sc=True,
            needs_layout_passes=False,
        ),
        scratch_types=[
            pltpu.VMEM((B_PER_W,), jnp.int32),
            pltpu.VMEM((B_PER_W,), jnp.int32),
            pltpu.VMEM((CHUNK, 8, DIMS), jnp.float32),
            pltpu.VMEM((CHUNK, 8, DIMS), jnp.float32),
            pltpu.VMEM((CHUNK // 2, VROW), jnp.float32),
            pltpu.SemaphoreType.DMA,
            pltpu.SemaphoreType.DMA,
        ],
    )
    packed = kfn(u, m, user_table)
    return packed.reshape(BATCH, DIMS)


def kernel(u, m, user_table, movie_table):
    return _gmf(u, m, user_table)


# speedup vs baseline: 12.2334x; 1.0478x over previous
import jax
import jax.numpy as jnp
from jax import lax
from jax.experimental import pallas as pl
from jax.experimental.pallas import tpu as pltpu
from jax.experimental.pallas import tpu_sc as plsc

BATCH = 16384
DIMS = 64
LANES = 16
VROW = 2 * DIMS

_info = plsc.get_sparse_core_info()
NC = _info.num_cores
NS = _info.num_subcores
NW = NC * NS

B_PER_W = BATCH // NW
CHUNK = 16
N_CHUNKS = B_PER_W // CHUNK


def _gmf_body(u_hbm, m_hbm, table_hbm, out_hbm,
              idx_u, idx_m, blk_u, blk_m, outb,
              sem_u0, sem_m0, sem_u1, sem_m1):
    wid = lax.axis_index("s") * NC + lax.axis_index("c")
    base = pl.multiple_of(wid * B_PER_W, B_PER_W)

    pltpu.sync_copy(u_hbm.at[pl.ds(base, B_PER_W)], idx_u)
    pltpu.sync_copy(m_hbm.at[pl.ds(base, B_PER_W)], idx_m)

    def issue(k, buf, sem_u, sem_m):
        def wave(c16, _):
            vu = idx_u[pl.ds(k * CHUNK + c16 * LANES, LANES)]
            vm = idx_m[pl.ds(k * CHUNK + c16 * LANES, LANES)]
            for l in range(LANES):
                c = c16 * LANES + l
                ru = pl.multiple_of(
                    lax.shift_left(lax.shift_right_logical(vu[l], 3), 3), 8)
                rm = pl.multiple_of(
                    lax.shift_left(lax.shift_right_logical(vm[l], 3), 3), 8)
                pltpu.async_copy(
                    table_hbm.at[pl.ds(ru, 8), :], blk_u.at[buf, c], sem_u)
                pltpu.async_copy(
                    table_hbm.at[pl.ds(rm, 8), :], blk_m.at[buf, c], sem_m)
            return 0

        lax.fori_loop(0, CHUNK // LANES, wave, 0)

    def drain(buf, sem_u, sem_m):
        def one(c, _):
            pltpu.make_async_copy(
                table_hbm.at[pl.ds(0, 8), :], blk_u.at[buf, c], sem_u).wait()
            pltpu.make_async_copy(
                table_hbm.at[pl.ds(0, 8), :], blk_m.at[buf, c], sem_m).wait()
            return 0

        lax.fori_loop(0, CHUNK, one, 0)

    def consume(k, buf):
        def mul16(c16, _):
            vu = idx_u[pl.ds(k * CHUNK + c16 * LANES, LANES)]
            vm = idx_m[pl.ds(k * CHUNK + c16 * LANES, LANES)]
            for l in range(LANES):
                b = c16 * LANES + l
                su = jnp.bitwise_and(vu[l], 7)
                sm = jnp.bitwise_and(vm[l], 7)
                v = b // 2
                p = b % 2
                for g in range(DIMS // LANES):
                    sl = pl.ds(g * LANES, LANES)
                    outb[v, pl.ds(p * DIMS + g * LANES, LANES)] = (
                        blk_u[buf, b, su, sl] * blk_m[buf, b, sm, sl])
            return 0

        lax.fori_loop(0, CHUNK // LANES, mul16, 0)

        pltpu.sync_copy(
            outb,
            out_hbm.at[pl.ds(pl.multiple_of((base + k * CHUNK) // 2, CHUNK // 2),
                             CHUNK // 2)])

    issue(0, 0, sem_u0, sem_m0)

    def chunk_pair(kk, _):
        k0 = kk * 2
        issue(k0 + 1, 1, sem_u1, sem_m1)
        drain(0, sem_u0, sem_m0)
        consume(k0, 0)

        @pl.when(k0 + 2 < N_CHUNKS)
        def _():
            issue(k0 + 2, 0, sem_u0, sem_m0)

        drain(1, sem_u1, sem_m1)
        consume(k0 + 1, 1)
        return 0

    lax.fori_loop(0, N_CHUNKS // 2, chunk_pair, 0)


@jax.jit
def _gmf(u, m, user_table):
    kfn = pl.kernel(
        _gmf_body,
        out_type=jax.ShapeDtypeStruct((BATCH // 2, VROW), jnp.float32),
        mesh=plsc.VectorSubcoreMesh(core_axis_name="c", subcore_axis_name="s"),
        compiler_params=pltpu.CompilerParams(
            use_tc_tiling_on_sc=True,
        ),
        scratch_types=[
            pltpu.VMEM((B_PER_W,), jnp.int32),
            pltpu.VMEM((B_PER_W,), jnp.int32),
            pltpu.VMEM((2, CHUNK, 8, DIMS), jnp.float32),
            pltpu.VMEM((2, CHUNK, 8, DIMS), jnp.float32),
            pltpu.VMEM((CHUNK // 2, VROW), jnp.float32),
            pltpu.SemaphoreType.DMA,
            pltpu.SemaphoreType.DMA,
            pltpu.SemaphoreType.DMA,
            pltpu.SemaphoreType.DMA,
        ],
    )
    packed = kfn(u, m, user_table)
    return packed.reshape(BATCH, DIMS)


def kernel(u, m, user_table, movie_table):
    return _gmf(u, m, user_table)
